# baseline (speedup 1.0000x reference)
import functools

import jax
import jax.numpy as jnp
from jax.experimental import pallas as pl

_BM = 512


def _ceil_to(n, m):
    return (n + m - 1) // m * m


def _pad_rows(x, mp):
    m = x.shape[0]
    if m == mp:
        return x
    return jnp.pad(x, ((0, mp - m),) + ((0, 0),) * (x.ndim - 1))


def _row_call(body, n_out_cols, out_dtype, *arrays):
    m = max(a.shape[0] for a in arrays)
    mp = _ceil_to(m, _BM)
    ins = []
    specs = []
    for a in arrays:
        if a.shape[0] == m:
            ins.append(_pad_rows(a, mp))
            specs.append(pl.BlockSpec((_BM,) + a.shape[1:],
                                      lambda i, nd=a.ndim: (i,) + (0,) * (nd - 1)))
        else:
            ins.append(a)
            specs.append(pl.BlockSpec(a.shape, lambda i, nd=a.ndim: (0,) * nd))
    out = pl.pallas_call(
        body,
        grid=(mp // _BM,),
        in_specs=specs,
        out_specs=pl.BlockSpec((_BM, n_out_cols), lambda i: (i, 0)),
        out_shape=jax.ShapeDtypeStruct((mp, n_out_cols), out_dtype),
    )(*ins)
    return out[:m]


def _mm(x, w, b=None, act=None, res=None):
    k = x.shape[1]
    n = w.shape[1]
    bb = jnp.zeros((1, n), x.dtype) if b is None else b.reshape(1, n)

    if res is None:
        def body(x_ref, w_ref, b_ref, o_ref):
            acc = jnp.dot(x_ref[...], w_ref[...], preferred_element_type=jnp.float32)
            acc = acc + b_ref[...]
            if act == "relu":
                acc = jnp.maximum(acc, 0.0)
            o_ref[...] = acc
        return _row_call(body, n, x.dtype, x, w, bb)

    def body(x_ref, w_ref, b_ref, r_ref, o_ref):
        acc = jnp.dot(x_ref[...], w_ref[...], preferred_element_type=jnp.float32)
        acc = acc + b_ref[...] + r_ref[...]
        if act == "relu":
            acc = jnp.maximum(acc, 0.0)
        o_ref[...] = acc
    return _row_call(body, n, x.dtype, x, w, bb, res)


def _lstm_step(q_star, h, c, wi, wh, b):
    d = h.shape[1]

    def body(q_ref, h_ref, c_ref, wi_ref, wh_ref, b_ref, h_out, c_out):
        g = (jnp.dot(q_ref[...], wi_ref[...], preferred_element_type=jnp.float32)
             + jnp.dot(h_ref[...], wh_ref[...], preferred_element_type=jnp.float32)
             + b_ref[...])
        ig = g[:, :d]
        fg = g[:, d:2 * d]
        gg = g[:, 2 * d:3 * d]
        og = g[:, 3 * d:]
        cn = jax.nn.sigmoid(fg) * c_ref[...] + jax.nn.sigmoid(ig) * jnp.tanh(gg)
        h_out[...] = jax.nn.sigmoid(og) * jnp.tanh(cn)
        c_out[...] = cn

    m = q_star.shape[0]
    mp = _ceil_to(m, _BM)
    n4 = 4 * d
    two = pl.pallas_call(
        body,
        grid=(mp // _BM,),
        in_specs=[
            pl.BlockSpec((_BM, 2 * d), lambda i: (i, 0)),
            pl.BlockSpec((_BM, d), lambda i: (i, 0)),
            pl.BlockSpec((_BM, d), lambda i: (i, 0)),
            pl.BlockSpec((2 * d, n4), lambda i: (0, 0)),
            pl.BlockSpec((d, n4), lambda i: (0, 0)),
            pl.BlockSpec((1, n4), lambda i: (0, 0)),
        ],
        out_specs=[
            pl.BlockSpec((_BM, d), lambda i: (i, 0)),
            pl.BlockSpec((_BM, d), lambda i: (i, 0)),
        ],
        out_shape=[
            jax.ShapeDtypeStruct((mp, d), q_star.dtype),
            jax.ShapeDtypeStruct((mp, d), q_star.dtype),
        ],
    )(_pad_rows(q_star, mp), _pad_rows(h, mp), _pad_rows(c, mp), wi, wh,
      b.reshape(1, n4))
    return two[0][:m], two[1][:m]


def _rowdot(x, q):
    def body(x_ref, q_ref, o_ref):
        o_ref[...] = jnp.sum(x_ref[...] * q_ref[...], axis=1, keepdims=True)
    return _row_call(body, 1, x.dtype, x, q)


def _exp_sub(e, m_sel):
    def body(e_ref, m_ref, o_ref):
        o_ref[...] = jnp.exp(e_ref[...] - m_ref[...])
    return _row_call(body, 1, e.dtype, e, m_sel)


def _scale_rows(ex, s_sel, x):
    def body(ex_ref, s_ref, x_ref, o_ref):
        o_ref[...] = (ex_ref[...] / (s_ref[...] + 1e-16)) * x_ref[...]
    return _row_call(body, x.shape[1], x.dtype, ex, s_sel, x)


def _div_clip(num, cnt):
    def body(n_ref, c_ref, o_ref):
        o_ref[...] = n_ref[...] / jnp.maximum(c_ref[...], 1.0)
    return _row_call(body, num.shape[1], num.dtype, num, cnt)


def _film(gamma, x, beta):
    def body(g_ref, x_ref, b_ref, o_ref):
        o_ref[...] = g_ref[...] * x_ref[...] + b_ref[...]
    return _row_call(body, x.shape[1], x.dtype, gamma, x, beta)


def _alpha_score(msg, att):
    def body(m_ref, a_ref, o_ref):
        s = jnp.sum(m_ref[...] * a_ref[...], axis=1, keepdims=True)
        o_ref[...] = jnp.where(s >= 0.0, s, 0.2 * s)
    return _row_call(body, 1, msg.dtype, msg, att.reshape(1, -1))


def _mask_rows(q_star, cnt):
    def body(q_ref, c_ref, o_ref):
        o_ref[...] = jnp.where(c_ref[...] > 0.0, q_ref[...], 0.0)
    return _row_call(body, q_star.shape[1], q_star.dtype, q_star, cnt)


def _seg_softmax_parts(e, ids, num):
    e1 = e[:, 0]
    m = jax.ops.segment_max(e1, ids, num_segments=num)
    m = jnp.where(jnp.isfinite(m), m, 0.0)
    ex = _exp_sub(e, m[ids][:, None])
    s = jax.ops.segment_sum(ex[:, 0], ids, num_segments=num)
    return ex, s[ids][:, None]


def _set2set(x, ids, num, wi, wh, b):
    d = x.shape[1]
    m = x.shape[0]
    h = jnp.zeros((num, d), x.dtype)
    c = jnp.zeros((num, d), x.dtype)
    q_star = jnp.zeros((num, 2 * d), x.dtype)
    for _ in range(2):
        h, c = _lstm_step(q_star, h, c, wi, wh, b)
        q_sel = h[ids]
        e = _rowdot(x, q_sel)
        ex, s_sel = _seg_softmax_parts(e, ids, num)
        ax = _scale_rows(ex, s_sel, x)
        r = jax.ops.segment_sum(ax, ids, num_segments=num)
        q_star = jnp.concatenate([h, r], axis=1)
    cnt = jax.ops.segment_sum(jnp.ones((m,), x.dtype), ids, num_segments=num)
    return _mask_rows(q_star, cnt[:, None]), cnt


def kernel(x_atom, x_group, cond_atom, atom_idx, group_idx, edge_index_group,
           W_a2g, b_a2g, Wi1, Wh1, bl1, W_merge, b_merge, Wg1, bg1, Wg2, bg2,
           Wb1, bb1, Wb2, bb2, W_gproj, b_gproj, W_msg, b_msg, W_self, b_self,
           att, Wi2, Wh2, bl2, W_g2a, b_g2a):
    Gm = x_group.shape[0]
    Na = x_atom.shape[0]

    xg_in = _mm(x_group[:, :40], W_gproj, b_gproj)

    xa_proj = _mm(x_atom, W_a2g, b_a2g)
    xa_items = xa_proj[atom_idx]
    xg_a2g, _ = _set2set(xa_items, group_idx, Gm, Wi1, Wh1, bl1)
    xg_from_atom = _mm(xg_a2g, W_merge, b_merge)

    cond_sel = cond_atom[atom_idx]
    cond_sum = jax.ops.segment_sum(cond_sel, group_idx, num_segments=Gm)
    cnt = jax.ops.segment_sum(jnp.ones((atom_idx.shape[0], 1), x_atom.dtype),
                              group_idx, num_segments=Gm)
    cond_g = _div_clip(cond_sum, cnt)

    gamma = _mm(_mm(cond_g, Wg1, bg1, act="relu"), Wg2, bg2)
    beta = _mm(_mm(cond_g, Wb1, bb1, act="relu"), Wb2, bb2)
    xg = _film(gamma, xg_from_atom, beta)

    src = edge_index_group[0]
    dst = edge_index_group[1]
    msg = _mm(xg[src], W_msg, b_msg)
    alpha = _alpha_score(msg, att)
    ex, s_sel = _seg_softmax_parts(alpha, dst, Gm)
    amsg = _scale_rows(ex, s_sel, msg)
    agg = jax.ops.segment_sum(amsg, dst, num_segments=Gm)
    xg = _mm(xg, W_self, b_self, res=agg, act="relu")
    xg = jnp.concatenate([xg_in, xg], axis=1)

    xg_items = xg[group_idx]
    xa_g2a, _ = _set2set(xg_items, atom_idx, Na, Wi2, Wh2, bl2)
    xa_out = _mm(xa_g2a, W_g2a, b_g2a, res=x_atom)
    return (xa_out, xg)
